# Initial kernel scaffold; baseline (speedup 1.0000x reference)
import jax
import jax.numpy as jnp
from jax.experimental import pallas as pl


def kernel(x, edge_index, edge_attr, edge_len_embbed, edge_len, Wq, Wk1, Wk2, Wv1, Wv2, Wdot):
    raise NotImplementedError("write your pallas kernel here")


# speedup vs baseline: 3.7041x; 3.7041x over previous
import jax
import jax.numpy as jnp
import numpy as np
from jax import lax
from jax.experimental import pallas as pl
from jax.experimental.pallas import tpu as pltpu
from jax.experimental.pallas import tpu_sc as plsc

N = 10000
E = 160000
D_IN = 128
MUL = 16
BASIS = 16
HID = 64
MAX_RADIUS = 3.15

NC = 2
NS = 16
NW = NC * NS
EPW = E // NW
GC = 200
SC_CHUNK = 200
NPAD = 10240
NPC = NPAD // NS

T = 1280
GRID = E // T


def _gather_body(src_hbm, dst_hbm, x_hbm, xg_hbm, xd_hbm,
                 idx_v, rows_v, sem):
    wid = lax.axis_index("s") * NC + lax.axis_index("c")
    base = wid * EPW

    def body(i, carry):
        off = base + i * GC
        pltpu.sync_copy(src_hbm.at[pl.ds(off, GC)], idx_v)
        pltpu.async_copy(x_hbm.at[idx_v], rows_v, sem).wait()
        pltpu.sync_copy(rows_v, xg_hbm.at[pl.ds(off, GC)])
        pltpu.sync_copy(dst_hbm.at[pl.ds(off, GC)], idx_v)
        pltpu.async_copy(x_hbm.at[idx_v], rows_v, sem).wait()
        pltpu.sync_copy(rows_v, xd_hbm.at[pl.ds(off, GC)])
        return carry

    lax.fori_loop(0, EPW // GC, body, 0)


def _edge_body(xg_ref, xd_ref, elen_ref, sh0_ref, el_ref,
               w2_ref, wk1_ref, wv1_ref, sel_ref, wqd_ref, o_ref):
    xi = xg_ref[...] * sh0_ref[...]
    a = jnp.dot(xi, w2_ref[...], preferred_element_type=jnp.float32)
    qdd = jnp.dot(xd_ref[...], wqd_ref[...], preferred_element_type=jnp.float32)
    emb = elen_ref[...]
    hk = jnp.dot(emb, wk1_ref[...], preferred_element_type=jnp.float32)
    hv = jnp.dot(emb, wv1_ref[...], preferred_element_type=jnp.float32)
    hk = hk * jax.nn.sigmoid(hk)
    hv = hv * jax.nn.sigmoid(hv)
    hk_t = jnp.tile(hk, (1, MUL))
    hv_t = jnp.tile(hv, (1, MUL))
    sel = sel_ref[...]
    k = jnp.dot(a[:, :MUL * HID] * hk_t, sel, preferred_element_type=jnp.float32)
    v = jnp.dot(a[:, MUL * HID:] * hv_t, sel, preferred_element_type=jnp.float32)
    logit = jnp.sum(k * qdd, axis=1, keepdims=True)
    t = 10.0 * (1.0 - el_ref[...] * (1.0 / MAX_RADIUS))
    cut = jnp.where(t > 0.0, jnp.exp(-1.0 / jnp.where(t > 0.0, t, 1.0)), 0.0)
    expw = cut * jnp.exp(logit)
    sv = jnp.sqrt(expw) * v
    o_ref[:, :MUL] = sv
    o_ref[:, MUL:MUL + 1] = expw
    o_ref[:, MUL + 1:] = jnp.zeros((T, D_IN - MUL - 1), jnp.float32)


def _scatter_body(dst_hbm, ewsv_hbm, zeros_hbm, acc_hbm,
                  idx_v, rows_v, shared):
    cid = lax.axis_index("c")
    sid = lax.axis_index("s")
    wid = sid * NC + cid
    pltpu.sync_copy(zeros_hbm.at[pl.ds(sid * NPC, NPC)],
                    shared.at[pl.ds(sid * NPC, NPC)])
    plsc.subcore_barrier()
    base = wid * EPW

    def body(i, carry):
        off = base + i * SC_CHUNK
        pltpu.sync_copy(dst_hbm.at[pl.ds(off, SC_CHUNK)], idx_v)
        pltpu.sync_copy(ewsv_hbm.at[pl.ds(off, SC_CHUNK)], rows_v)
        pltpu.sync_copy(rows_v, shared.at[idx_v], add=True)
        return carry

    lax.fori_loop(0, EPW // SC_CHUNK, body, 0)
    plsc.subcore_barrier()
    pltpu.sync_copy(shared.at[pl.ds(sid * NPC, NPC)],
                    acc_hbm.at[cid, pl.ds(sid * NPC, NPC)])


def _final_body(acc_ref, o_ref):
    p = acc_ref[0, :N] + acc_ref[1, :N]
    z = p[:, MUL:MUL + 1]
    o_ref[...] = p[:, :MUL] * lax.rsqrt(jnp.maximum(z, 1e-30))


def kernel(x, edge_index, edge_attr, edge_len_embbed, edge_len,
           Wq, Wk1, Wk2, Wv1, Wv2, Wdot):
    f32 = jnp.float32
    w2k = Wk2.reshape(HID, D_IN, MUL).transpose(1, 2, 0).reshape(D_IN, MUL * HID)
    w2v = Wv2.reshape(HID, D_IN, MUL).transpose(1, 2, 0).reshape(D_IN, MUL * HID)
    w2 = jnp.concatenate([w2k, w2v], axis=1) * (1.0 / (np.sqrt(HID) * np.sqrt(D_IN)))
    wk1 = Wk1 * (1.0 / np.sqrt(BASIS))
    wv1 = Wv1 * (1.0 / np.sqrt(BASIS))
    wq = Wq * (1.0 / (np.sqrt(D_IN) * float(MUL)))
    sel = (jnp.arange(MUL * HID, dtype=jnp.int32)[:, None] // HID
           == jnp.arange(MUL, dtype=jnp.int32)[None, :]).astype(f32)
    sh0 = edge_attr[:, 0:1]
    el = edge_len[:, None]
    zeros = jnp.zeros((NPAD, D_IN), f32)
    src = edge_index[0]
    dst = edge_index[1]

    mesh = plsc.VectorSubcoreMesh(core_axis_name="c", subcore_axis_name="s")
    gather = pl.kernel(
        _gather_body,
        out_type=[jax.ShapeDtypeStruct((E, D_IN), f32),
                  jax.ShapeDtypeStruct((E, D_IN), f32)],
        mesh=mesh,
        scratch_types=[pltpu.VMEM((GC,), jnp.int32),
                       pltpu.VMEM((GC, D_IN), f32),
                       pltpu.SemaphoreType.DMA],
    )
    xg, xd = gather(src, dst, x)

    ewsv = pl.pallas_call(
        _edge_body,
        grid=(GRID,),
        in_specs=[
            pl.BlockSpec((T, D_IN), lambda i: (i, 0)),
            pl.BlockSpec((T, D_IN), lambda i: (i, 0)),
            pl.BlockSpec((T, BASIS), lambda i: (i, 0)),
            pl.BlockSpec((T, 1), lambda i: (i, 0)),
            pl.BlockSpec((T, 1), lambda i: (i, 0)),
            pl.BlockSpec((D_IN, 2 * MUL * HID), lambda i: (0, 0)),
            pl.BlockSpec((BASIS, HID), lambda i: (0, 0)),
            pl.BlockSpec((BASIS, HID), lambda i: (0, 0)),
            pl.BlockSpec((MUL * HID, MUL), lambda i: (0, 0)),
            pl.BlockSpec((D_IN, MUL), lambda i: (0, 0)),
        ],
        out_specs=pl.BlockSpec((T, D_IN), lambda i: (i, 0)),
        out_shape=jax.ShapeDtypeStruct((E, D_IN), f32),
    )(xg, xd, edge_len_embbed, sh0, el, w2, wk1, wv1, sel,
      jnp.dot(wq, Wdot))

    scatter = pl.kernel(
        _scatter_body,
        out_type=jax.ShapeDtypeStruct((NC, NPAD, D_IN), f32),
        mesh=mesh,
        scratch_types=[pltpu.VMEM((SC_CHUNK,), jnp.int32),
                       pltpu.VMEM((SC_CHUNK, D_IN), f32),
                       pltpu.VMEM_SHARED((NPAD, D_IN), f32)],
    )
    acc = scatter(dst, ewsv, zeros)

    out = pl.pallas_call(
        _final_body,
        out_shape=jax.ShapeDtypeStruct((N, MUL), f32),
    )(acc)
    return out


# speedup vs baseline: 4.1761x; 1.1274x over previous
import functools

import jax
import jax.numpy as jnp
import numpy as np
from jax import lax
from jax.experimental import pallas as pl
from jax.experimental.pallas import tpu as pltpu
from jax.experimental.pallas import tpu_sc as plsc

N = 10000
E = 160000
D_IN = 128
MUL = 16
BASIS = 16
HID = 64
MAX_RADIUS = 3.15

NC = 2
NS = 16
NW = NC * NS
SC_CHUNK = 40
NPAD = 10240
NPC = NPAD // NS

OW = 128
T = 1280

EH0 = 81920
EH1 = E - EH0
GC0 = 160
GC1 = 40


def _gather_body(epw, gc, src_hbm, dst_hbm, x_hbm, xg_hbm, xd_hbm,
                 isrc_v, idst_v, ga, gb, da, db, gsema, gsemb, osema, osemb):
    wid = lax.axis_index("s") * NC + lax.axis_index("c")
    base = wid * epw
    pltpu.sync_copy(src_hbm.at[pl.ds(base, epw)], isrc_v)
    pltpu.sync_copy(dst_hbm.at[pl.ds(base, epw)], idst_v)

    def start_g(i, gbuf, dbuf, sem):
        pltpu.async_copy(x_hbm.at[isrc_v.at[pl.ds(i * gc, gc)]], gbuf, sem)
        pltpu.async_copy(x_hbm.at[idst_v.at[pl.ds(i * gc, gc)]], dbuf, sem)

    def wait_g(gbuf, dbuf, sem):
        pltpu.make_async_copy(x_hbm.at[pl.ds(0, gc)], gbuf, sem).wait()
        pltpu.make_async_copy(x_hbm.at[pl.ds(0, gc)], dbuf, sem).wait()

    def start_o(i, gbuf, dbuf, sem):
        off = base + i * gc
        pltpu.async_copy(gbuf, xg_hbm.at[pl.ds(off, gc)], sem)
        pltpu.async_copy(dbuf, xd_hbm.at[pl.ds(off, gc)], sem)

    def wait_o(gbuf, dbuf, sem):
        pltpu.make_async_copy(gbuf, xg_hbm.at[pl.ds(base, gc)], sem).wait()
        pltpu.make_async_copy(dbuf, xd_hbm.at[pl.ds(base, gc)], sem).wait()

    n = epw // gc
    start_g(0, ga, da, gsema)

    def body(j, carry):
        i = 2 * j
        start_g(i + 1, gb, db, gsemb)
        wait_g(ga, da, gsema)
        start_o(i, ga, da, osema)
        wait_g(gb, db, gsemb)
        start_o(i + 1, gb, db, osemb)
        wait_o(ga, da, osema)
        start_g(i + 2, ga, da, gsema)
        wait_o(gb, db, osemb)
        return carry

    if n % 2 == 1:
        lax.fori_loop(0, (n - 1) // 2, body, 0)
        wait_g(ga, da, gsema)
        start_o(n - 1, ga, da, osema)
        wait_o(ga, da, osema)
    else:
        lax.fori_loop(0, (n - 2) // 2, body, 0)
        start_g(n - 1, gb, db, gsemb)
        wait_g(ga, da, gsema)
        start_o(n - 2, ga, da, osema)
        wait_g(gb, db, gsemb)
        start_o(n - 1, gb, db, osemb)
        wait_o(ga, da, osema)
        wait_o(gb, db, osemb)


def _edge_body(xg_ref, xd_ref, elen_ref, sh0_ref, el_ref,
               w2_ref, wk1_ref, wv1_ref, sel_ref, wqd_ref, o_ref,
               vscr, lscr):
    t = 10.0 * (1.0 - el_ref[...].reshape(T) * (1.0 / MAX_RADIUS))
    pos = t > 0.0
    rt = 1.0 / jnp.where(pos, t, 1.0)
    logit = lscr[...].reshape(T)
    v_prev = vscr[...]
    expw = jnp.where(pos, jnp.exp(logit - rt), 0.0)
    s = jnp.where(pos, jnp.exp(0.5 * logit - 0.5 * rt), 0.0)
    sv = s[:, None] * v_prev
    o_ref[:, :MUL] = sv
    o_ref[:, MUL:MUL + 1] = expw[:, None]
    o_ref[:, MUL + 1:] = jnp.zeros((T, OW - MUL - 1), jnp.float32)
    xi = xg_ref[...] * sh0_ref[...]
    a = jnp.dot(xi, w2_ref[...], preferred_element_type=jnp.float32)
    qdd = jnp.dot(xd_ref[...], wqd_ref[...], preferred_element_type=jnp.float32)
    emb = elen_ref[...]
    hk = jnp.dot(emb, wk1_ref[...], preferred_element_type=jnp.float32)
    hv = jnp.dot(emb, wv1_ref[...], preferred_element_type=jnp.float32)
    hk = hk * jax.nn.sigmoid(hk)
    hv = hv * jax.nn.sigmoid(hv)
    hk_t = jnp.tile(hk, (1, MUL))
    hv_t = jnp.tile(hv, (1, MUL))
    sel = sel_ref[...]
    k = jnp.dot(a[:, :MUL * HID] * hk_t, sel, preferred_element_type=jnp.float32)
    v = jnp.dot(a[:, MUL * HID:] * hv_t, sel, preferred_element_type=jnp.float32)
    vscr[...] = v
    lscr[...] = jnp.sum(k * qdd, axis=1, keepdims=True)


def _scatter_body(epw, dst_hbm, ewsv_hbm, zeros_hbm, acc_hbm,
                  idx_v, rows_a, rows_b, shared, lsema, lsemb):
    cid = lax.axis_index("c")
    sid = lax.axis_index("s")
    wid = sid * NC + cid
    pltpu.sync_copy(zeros_hbm.at[pl.ds(sid * NPC, NPC)],
                    shared.at[pl.ds(sid * NPC, NPC)])
    base = wid * epw
    nck = epw // SC_CHUNK
    pltpu.sync_copy(dst_hbm.at[pl.ds(wid * nck, nck)], idx_v)
    plsc.subcore_barrier()

    def start_l(i, buf, sem):
        pltpu.async_copy(ewsv_hbm.at[pl.ds(base + i * SC_CHUNK, SC_CHUNK)],
                         buf, sem)

    def wait_l(buf, sem):
        pltpu.make_async_copy(ewsv_hbm.at[pl.ds(base, SC_CHUNK)],
                              buf, sem).wait()

    def scat(i, buf):
        pltpu.sync_copy(buf, shared.at[idx_v.at[i, 0]], add=True)

    start_l(0, rows_a, lsema)

    def body(j, carry):
        i = 2 * j
        start_l(i + 1, rows_b, lsemb)
        wait_l(rows_a, lsema)
        scat(i, rows_a)
        start_l(i + 2, rows_a, lsema)
        wait_l(rows_b, lsemb)
        scat(i + 1, rows_b)
        return carry

    if nck % 2 == 1:
        lax.fori_loop(0, (nck - 1) // 2, body, 0)
        wait_l(rows_a, lsema)
        scat(nck - 1, rows_a)
    else:
        lax.fori_loop(0, (nck - 2) // 2, body, 0)
        start_l(nck - 1, rows_b, lsemb)
        wait_l(rows_a, lsema)
        scat(nck - 2, rows_a)
        wait_l(rows_b, lsemb)
        scat(nck - 1, rows_b)
    plsc.subcore_barrier()
    pltpu.sync_copy(shared.at[pl.ds(sid * NPC, NPC)],
                    acc_hbm.at[cid, pl.ds(sid * NPC, NPC)])


def _final_body(a0_ref, a1_ref, o_ref):
    p = (a0_ref[0, :N] + a0_ref[1, :N]
         + a1_ref[0, :N] + a1_ref[1, :N])
    z = p[:, MUL:MUL + 1]
    o_ref[...] = p[:, :MUL] * lax.rsqrt(jnp.maximum(z, 1e-30))


def kernel(x, edge_index, edge_attr, edge_len_embbed, edge_len,
           Wq, Wk1, Wk2, Wv1, Wv2, Wdot):
    f32 = jnp.float32
    w2k = Wk2.reshape(HID, D_IN, MUL).transpose(1, 2, 0).reshape(D_IN, MUL * HID)
    w2v = Wv2.reshape(HID, D_IN, MUL).transpose(1, 2, 0).reshape(D_IN, MUL * HID)
    w2 = jnp.concatenate([w2k, w2v], axis=1) * (1.0 / (np.sqrt(HID) * np.sqrt(D_IN)))
    wk1 = Wk1 * (1.0 / np.sqrt(BASIS))
    wv1 = Wv1 * (1.0 / np.sqrt(BASIS))
    wqd = jnp.dot(Wq * (1.0 / (np.sqrt(D_IN) * float(MUL))), Wdot)
    sel = (jnp.arange(MUL * HID, dtype=jnp.int32)[:, None] // HID
           == jnp.arange(MUL, dtype=jnp.int32)[None, :]).astype(f32)
    sh0 = edge_attr[:, 0:1]
    el = edge_len[:, None]
    zeros = jnp.zeros((NPAD, OW), f32)
    src = edge_index[0]
    dst = edge_index[1]

    mesh = plsc.VectorSubcoreMesh(core_axis_name="c", subcore_axis_name="s")

    def make_gather(eh, gc):
        epw = eh // NW
        return pl.kernel(
            functools.partial(_gather_body, epw, gc),
            out_type=[jax.ShapeDtypeStruct((eh, D_IN), f32),
                      jax.ShapeDtypeStruct((eh, D_IN), f32)],
            mesh=mesh,
            scratch_types=[pltpu.VMEM((epw,), jnp.int32),
                           pltpu.VMEM((epw,), jnp.int32),
                           pltpu.VMEM((gc, D_IN), f32),
                           pltpu.VMEM((gc, D_IN), f32),
                           pltpu.VMEM((gc, D_IN), f32),
                           pltpu.VMEM((gc, D_IN), f32),
                           pltpu.SemaphoreType.DMA,
                           pltpu.SemaphoreType.DMA,
                           pltpu.SemaphoreType.DMA,
                           pltpu.SemaphoreType.DMA],
        )

    def make_edge(eh):
        grid = eh // T
        cur = lambda i: (jnp.minimum(i, grid - 1), 0)
        prev = lambda i: (jnp.maximum(i, 1) - 1, 0)
        return pl.pallas_call(
            _edge_body,
            grid=(grid + 1,),
            in_specs=[
                pl.BlockSpec((T, D_IN), cur),
                pl.BlockSpec((T, D_IN), cur),
                pl.BlockSpec((T, BASIS), cur),
                pl.BlockSpec((T, 1), cur),
                pl.BlockSpec((T, 1), prev),
                pl.BlockSpec((D_IN, 2 * MUL * HID), lambda i: (0, 0)),
                pl.BlockSpec((BASIS, HID), lambda i: (0, 0)),
                pl.BlockSpec((BASIS, HID), lambda i: (0, 0)),
                pl.BlockSpec((MUL * HID, MUL), lambda i: (0, 0)),
                pl.BlockSpec((D_IN, MUL), lambda i: (0, 0)),
            ],
            out_specs=pl.BlockSpec((T, OW), prev),
            out_shape=jax.ShapeDtypeStruct((eh, OW), f32),
            scratch_shapes=[pltpu.VMEM((T, MUL), f32),
                            pltpu.VMEM((T, 1), f32)],
        )

    def make_scatter(eh):
        epw = eh // NW
        return pl.kernel(
            functools.partial(_scatter_body, epw),
            out_type=jax.ShapeDtypeStruct((NC, NPAD, OW), f32),
            mesh=mesh,
            scratch_types=[pltpu.VMEM((epw // SC_CHUNK, 1, SC_CHUNK), jnp.int32),
                           pltpu.VMEM((SC_CHUNK, OW), f32),
                           pltpu.VMEM((SC_CHUNK, OW), f32),
                           pltpu.VMEM_SHARED((NPAD, OW), f32),
                           pltpu.SemaphoreType.DMA,
                           pltpu.SemaphoreType.DMA],
        )

    halves = []
    for (lo, eh, gc) in ((0, EH0, GC0), (EH0, EH1, GC1)):
        xg, xd = make_gather(eh, gc)(src[lo:lo + eh], dst[lo:lo + eh], x)
        halves.append((lo, eh, xg, xd))

    accs = []
    for (lo, eh, xg, xd) in halves:
        ewsv = make_edge(eh)(xg, xd, edge_len_embbed[lo:lo + eh],
                             sh0[lo:lo + eh], el[lo:lo + eh],
                             w2, wk1, wv1, sel, wqd)
        acc = make_scatter(eh)(
            dst[lo:lo + eh].reshape(eh // SC_CHUNK, 1, SC_CHUNK), ewsv, zeros)
        accs.append(acc)

    out = pl.pallas_call(
        _final_body,
        out_shape=jax.ShapeDtypeStruct((N, MUL), f32),
    )(accs[0], accs[1])
    return out


# speedup vs baseline: 4.6866x; 1.1223x over previous
import jax
import jax.numpy as jnp
import numpy as np
from jax import lax
from jax.experimental import pallas as pl
from jax.experimental.pallas import tpu as pltpu
from jax.experimental.pallas import tpu_sc as plsc

N = 10000
E = 160000
D_IN = 128
MUL = 16
BASIS = 16
HID = 64
MAX_RADIUS = 3.15

NC = 2
NS = 16
NW = NC * NS
EPW = E // NW
GC = 200
SC_CHUNK = 40
NPAD = 10240
NPC = NPAD // NS

OW = 128
T = 2000
GRID = E // T


def _gather_body(src_hbm, dst_hbm, x_hbm, xg_hbm, xd_hbm,
                 isrc_v, idst_v, ga, gb, da, db, gsema, gsemb, osema, osemb):
    wid = lax.axis_index("s") * NC + lax.axis_index("c")
    base = wid * EPW
    pltpu.sync_copy(src_hbm.at[pl.ds(base, EPW)], isrc_v)
    pltpu.sync_copy(dst_hbm.at[pl.ds(base, EPW)], idst_v)

    def start_g(i, gbuf, dbuf, sem):
        pltpu.async_copy(x_hbm.at[isrc_v.at[pl.ds(i * GC, GC)]], gbuf, sem)
        pltpu.async_copy(x_hbm.at[idst_v.at[pl.ds(i * GC, GC)]], dbuf, sem)

    def wait_g(gbuf, dbuf, sem):
        pltpu.make_async_copy(x_hbm.at[pl.ds(0, GC)], gbuf, sem).wait()
        pltpu.make_async_copy(x_hbm.at[pl.ds(0, GC)], dbuf, sem).wait()

    def start_o(i, gbuf, dbuf, sem):
        off = base + i * GC
        pltpu.async_copy(gbuf, xg_hbm.at[pl.ds(off, GC)], sem)
        pltpu.async_copy(dbuf, xd_hbm.at[pl.ds(off, GC)], sem)

    def wait_o(gbuf, dbuf, sem):
        pltpu.make_async_copy(gbuf, xg_hbm.at[pl.ds(base, GC)], sem).wait()
        pltpu.make_async_copy(dbuf, xd_hbm.at[pl.ds(base, GC)], sem).wait()

    start_g(0, ga, da, gsema)

    def body(j, carry):
        i = 2 * j
        start_g(i + 1, gb, db, gsemb)
        wait_g(ga, da, gsema)
        start_o(i, ga, da, osema)
        wait_g(gb, db, gsemb)
        start_o(i + 1, gb, db, osemb)
        wait_o(ga, da, osema)
        start_g(i + 2, ga, da, gsema)
        wait_o(gb, db, osemb)
        return carry

    lax.fori_loop(0, (EPW // GC) // 2, body, 0)
    wait_g(ga, da, gsema)
    start_o(EPW // GC - 1, ga, da, osema)
    wait_o(ga, da, osema)


def _edge_body(xg_ref, xd_ref, elen_ref, sh0_ref, el_ref,
               w2_ref, wkv1_ref, sel_ref, wqd_ref, o_ref,
               vscr, lscr):
    t = 10.0 * (1.0 - el_ref[...].reshape(T) * (1.0 / MAX_RADIUS))
    pos = t > 0.0
    rt = 1.0 / jnp.where(pos, t, 1.0)
    logit = lscr[...].reshape(T)
    v_prev = vscr[...]
    expw = jnp.where(pos, jnp.exp(logit - rt), 0.0)
    s = jnp.where(pos, jnp.exp(0.5 * logit - 0.5 * rt), 0.0)
    sv = s[:, None] * v_prev
    o_ref[:, :MUL] = sv
    o_ref[:, MUL:MUL + 1] = expw[:, None]
    o_ref[:, MUL + 1:] = jnp.zeros((T, OW - MUL - 1), jnp.float32)
    xi = xg_ref[...] * sh0_ref[...]
    a = jnp.dot(xi, w2_ref[...], preferred_element_type=jnp.float32)
    qdd = jnp.dot(xd_ref[...], wqd_ref[...], preferred_element_type=jnp.float32)
    emb = elen_ref[...]
    h2 = jnp.dot(emb, wkv1_ref[...], preferred_element_type=jnp.float32)
    h2 = h2 * jax.nn.sigmoid(h2)
    hk_t = jnp.tile(h2[:, :HID], (1, MUL))
    hv_t = jnp.tile(h2[:, HID:], (1, MUL))
    sel = sel_ref[...]
    k = jnp.dot(a[:, :MUL * HID] * hk_t, sel, preferred_element_type=jnp.float32)
    v = jnp.dot(a[:, MUL * HID:] * hv_t, sel, preferred_element_type=jnp.float32)
    vscr[...] = v
    lscr[...] = jnp.sum(k * qdd, axis=1, keepdims=True)


def _scatter_body(dst_hbm, ewsv_hbm, zeros_hbm, acc_hbm,
                  idx_v, rows_a, rows_b, shared, lsema, lsemb):
    cid = lax.axis_index("c")
    sid = lax.axis_index("s")
    wid = sid * NC + cid
    pltpu.sync_copy(zeros_hbm.at[pl.ds(sid * NPC, NPC)],
                    shared.at[pl.ds(sid * NPC, NPC)])
    base = wid * EPW
    nck = EPW // SC_CHUNK
    pltpu.sync_copy(dst_hbm.at[pl.ds(wid * nck, nck)], idx_v)
    plsc.subcore_barrier()

    def start_l(i, buf, sem):
        pltpu.async_copy(ewsv_hbm.at[pl.ds(base + i * SC_CHUNK, SC_CHUNK)],
                         buf, sem)

    def wait_l(buf, sem):
        pltpu.make_async_copy(ewsv_hbm.at[pl.ds(base, SC_CHUNK)],
                              buf, sem).wait()

    def scat(i, buf):
        pltpu.sync_copy(buf, shared.at[idx_v.at[i, 0]], add=True)

    start_l(0, rows_a, lsema)

    def body(j, carry):
        i = 2 * j
        start_l(i + 1, rows_b, lsemb)
        wait_l(rows_a, lsema)
        scat(i, rows_a)
        start_l(i + 2, rows_a, lsema)
        wait_l(rows_b, lsemb)
        scat(i + 1, rows_b)
        return carry

    lax.fori_loop(0, (EPW // SC_CHUNK) // 2, body, 0)
    wait_l(rows_a, lsema)
    scat(EPW // SC_CHUNK - 1, rows_a)
    plsc.subcore_barrier()
    pltpu.sync_copy(shared.at[pl.ds(sid * NPC, NPC)],
                    acc_hbm.at[cid, pl.ds(sid * NPC, NPC)])


def _final_body(acc_ref, o_ref):
    p = acc_ref[0, :N] + acc_ref[1, :N]
    z = p[:, MUL:MUL + 1]
    o_ref[...] = p[:, :MUL] * lax.rsqrt(jnp.maximum(z, 1e-30))


def kernel(x, edge_index, edge_attr, edge_len_embbed, edge_len,
           Wq, Wk1, Wk2, Wv1, Wv2, Wdot):
    f32 = jnp.float32
    w2k = Wk2.reshape(HID, D_IN, MUL).transpose(1, 2, 0).reshape(D_IN, MUL * HID)
    w2v = Wv2.reshape(HID, D_IN, MUL).transpose(1, 2, 0).reshape(D_IN, MUL * HID)
    w2 = jnp.concatenate([w2k, w2v], axis=1) * (1.0 / (np.sqrt(HID) * np.sqrt(D_IN)))
    wk1 = Wk1 * (1.0 / np.sqrt(BASIS))
    wv1 = Wv1 * (1.0 / np.sqrt(BASIS))
    wq = Wq * (1.0 / (np.sqrt(D_IN) * float(MUL)))
    sel = (jnp.arange(MUL * HID, dtype=jnp.int32)[:, None] // HID
           == jnp.arange(MUL, dtype=jnp.int32)[None, :]).astype(f32)
    sh0 = edge_attr[:, 0:1]
    el = edge_len[:, None]
    zeros = jnp.zeros((NPAD, OW), f32)
    src = edge_index[0]
    dst = edge_index[1]

    mesh = plsc.VectorSubcoreMesh(core_axis_name="c", subcore_axis_name="s")
    gather = pl.kernel(
        _gather_body,
        out_type=[jax.ShapeDtypeStruct((E, D_IN), f32),
                  jax.ShapeDtypeStruct((E, D_IN), f32)],
        mesh=mesh,
        scratch_types=[pltpu.VMEM((EPW,), jnp.int32),
                       pltpu.VMEM((EPW,), jnp.int32),
                       pltpu.VMEM((GC, D_IN), f32),
                       pltpu.VMEM((GC, D_IN), f32),
                       pltpu.VMEM((GC, D_IN), f32),
                       pltpu.VMEM((GC, D_IN), f32),
                       pltpu.SemaphoreType.DMA,
                       pltpu.SemaphoreType.DMA,
                       pltpu.SemaphoreType.DMA,
                       pltpu.SemaphoreType.DMA],
    )
    xg, xd = gather(src, dst, x)

    cur = lambda i: (jnp.minimum(i, GRID - 1), 0)
    prev = lambda i: (jnp.maximum(i, 1) - 1, 0)
    ewsv = pl.pallas_call(
        _edge_body,
        grid=(GRID + 1,),
        in_specs=[
            pl.BlockSpec((T, D_IN), cur),
            pl.BlockSpec((T, D_IN), cur),
            pl.BlockSpec((T, BASIS), cur),
            pl.BlockSpec((T, 1), cur),
            pl.BlockSpec((T, 1), prev),
            pl.BlockSpec((D_IN, 2 * MUL * HID), lambda i: (0, 0)),
            pl.BlockSpec((BASIS, 2 * HID), lambda i: (0, 0)),
            pl.BlockSpec((MUL * HID, MUL), lambda i: (0, 0)),
            pl.BlockSpec((D_IN, MUL), lambda i: (0, 0)),
        ],
        out_specs=pl.BlockSpec((T, OW), prev),
        out_shape=jax.ShapeDtypeStruct((E, OW), f32),
        scratch_shapes=[pltpu.VMEM((T, MUL), f32),
                        pltpu.VMEM((T, 1), f32)],
    )(xg, xd, edge_len_embbed, sh0, el, w2,
      jnp.concatenate([wk1, wv1], axis=1), sel, jnp.dot(wq, Wdot))

    scatter = pl.kernel(
        _scatter_body,
        out_type=jax.ShapeDtypeStruct((NC, NPAD, OW), f32),
        mesh=mesh,
        scratch_types=[pltpu.VMEM((EPW // SC_CHUNK, 1, SC_CHUNK), jnp.int32),
                       pltpu.VMEM((SC_CHUNK, OW), f32),
                       pltpu.VMEM((SC_CHUNK, OW), f32),
                       pltpu.VMEM_SHARED((NPAD, OW), f32),
                       pltpu.SemaphoreType.DMA,
                       pltpu.SemaphoreType.DMA],
    )
    acc = scatter(dst.reshape(E // SC_CHUNK, 1, SC_CHUNK), ewsv, zeros)

    out = pl.pallas_call(
        _final_body,
        out_shape=jax.ShapeDtypeStruct((N, MUL), f32),
    )(acc)
    return out


# speedup vs baseline: 4.7252x; 1.0082x over previous
import jax
import jax.numpy as jnp
import numpy as np
from jax import lax
from jax.experimental import pallas as pl
from jax.experimental.pallas import tpu as pltpu
from jax.experimental.pallas import tpu_sc as plsc

N = 10000
E = 160000
D_IN = 128
MUL = 16
BASIS = 16
HID = 64
MAX_RADIUS = 3.15

NC = 2
NS = 16
NW = NC * NS
EPW = E // NW
GC = 200
SC_CHUNK = 40
NPAD = 10240
NPC = NPAD // NS

OW = 128
T = 3200
GRID = E // T


def _gather_body(src_hbm, dst_hbm, x_hbm, xg_hbm, xd_hbm,
                 isrc_v, idst_v, ga, gb, da, db, gsema, gsemb, osema, osemb):
    wid = lax.axis_index("s") * NC + lax.axis_index("c")
    base = wid * EPW
    pltpu.sync_copy(src_hbm.at[pl.ds(base, EPW)], isrc_v)
    pltpu.sync_copy(dst_hbm.at[pl.ds(base, EPW)], idst_v)

    def start_g(i, gbuf, dbuf, sem):
        pltpu.async_copy(x_hbm.at[isrc_v.at[pl.ds(i * GC, GC)]], gbuf, sem)
        pltpu.async_copy(x_hbm.at[idst_v.at[pl.ds(i * GC, GC)]], dbuf, sem)

    def wait_g(gbuf, dbuf, sem):
        pltpu.make_async_copy(x_hbm.at[pl.ds(0, GC)], gbuf, sem).wait()
        pltpu.make_async_copy(x_hbm.at[pl.ds(0, GC)], dbuf, sem).wait()

    def start_o(i, gbuf, dbuf, sem):
        off = base + i * GC
        pltpu.async_copy(gbuf, xg_hbm.at[pl.ds(off, GC)], sem)
        pltpu.async_copy(dbuf, xd_hbm.at[pl.ds(off, GC)], sem)

    def wait_o(gbuf, dbuf, sem):
        pltpu.make_async_copy(gbuf, xg_hbm.at[pl.ds(base, GC)], sem).wait()
        pltpu.make_async_copy(dbuf, xd_hbm.at[pl.ds(base, GC)], sem).wait()

    start_g(0, ga, da, gsema)

    def body(j, carry):
        i = 2 * j
        start_g(i + 1, gb, db, gsemb)
        wait_g(ga, da, gsema)
        start_o(i, ga, da, osema)
        wait_g(gb, db, gsemb)
        start_o(i + 1, gb, db, osemb)
        wait_o(ga, da, osema)
        start_g(i + 2, ga, da, gsema)
        wait_o(gb, db, osemb)
        return carry

    lax.fori_loop(0, (EPW // GC) // 2, body, 0)
    wait_g(ga, da, gsema)
    start_o(EPW // GC - 1, ga, da, osema)
    wait_o(ga, da, osema)


def _edge_body(xg_ref, xd_ref, elen_ref, sh0_ref, el_ref,
               w2_ref, wkv1_ref, sel_ref, wqd_ref, o_ref,
               vscr, lscr):
    t = 10.0 * (1.0 - el_ref[...].reshape(T) * (1.0 / MAX_RADIUS))
    pos = t > 0.0
    rt = 1.0 / jnp.where(pos, t, 1.0)
    logit = lscr[...].reshape(T)
    v_prev = vscr[...]
    expw = jnp.where(pos, jnp.exp(logit - rt), 0.0)
    s = jnp.where(pos, jnp.exp(0.5 * logit - 0.5 * rt), 0.0)
    sv = s[:, None] * v_prev
    o_ref[:, :MUL] = sv
    o_ref[:, MUL:MUL + 1] = expw[:, None]
    o_ref[:, MUL + 1:] = jnp.zeros((T, OW - MUL - 1), jnp.float32)
    xi = xg_ref[...] * sh0_ref[...]
    a = jnp.dot(xi, w2_ref[...], preferred_element_type=jnp.float32)
    qdd = jnp.dot(xd_ref[...], wqd_ref[...], preferred_element_type=jnp.float32)
    emb = elen_ref[...]
    h2 = jnp.dot(emb, wkv1_ref[...], preferred_element_type=jnp.float32)
    h2 = h2 * jax.nn.sigmoid(h2)
    hk_t = jnp.tile(h2[:, :HID], (1, MUL))
    hv_t = jnp.tile(h2[:, HID:], (1, MUL))
    sel = sel_ref[...]
    k = jnp.dot(a[:, :MUL * HID] * hk_t, sel, preferred_element_type=jnp.float32)
    v = jnp.dot(a[:, MUL * HID:] * hv_t, sel, preferred_element_type=jnp.float32)
    vscr[...] = v
    lscr[...] = jnp.sum(k * qdd, axis=1, keepdims=True)


def _scatter_body(dst_hbm, ewsv_hbm, zeros_hbm, acc_hbm,
                  idx_v, rows_a, rows_b, shared, lsema, lsemb):
    cid = lax.axis_index("c")
    sid = lax.axis_index("s")
    wid = sid * NC + cid
    pltpu.sync_copy(zeros_hbm.at[pl.ds(sid * NPC, NPC)],
                    shared.at[pl.ds(sid * NPC, NPC)])
    base = wid * EPW
    nck = EPW // SC_CHUNK
    pltpu.sync_copy(dst_hbm.at[pl.ds(wid * nck, nck)], idx_v)
    plsc.subcore_barrier()

    def start_l(i, buf, sem):
        pltpu.async_copy(ewsv_hbm.at[pl.ds(base + i * SC_CHUNK, SC_CHUNK)],
                         buf, sem)

    def wait_l(buf, sem):
        pltpu.make_async_copy(ewsv_hbm.at[pl.ds(base, SC_CHUNK)],
                              buf, sem).wait()

    def scat(i, buf):
        pltpu.sync_copy(buf, shared.at[idx_v.at[i, 0]], add=True)

    start_l(0, rows_a, lsema)

    def body(j, carry):
        i = 2 * j
        start_l(i + 1, rows_b, lsemb)
        wait_l(rows_a, lsema)
        scat(i, rows_a)
        start_l(i + 2, rows_a, lsema)
        wait_l(rows_b, lsemb)
        scat(i + 1, rows_b)
        return carry

    lax.fori_loop(0, (EPW // SC_CHUNK) // 2, body, 0)
    wait_l(rows_a, lsema)
    scat(EPW // SC_CHUNK - 1, rows_a)
    plsc.subcore_barrier()
    pltpu.sync_copy(shared.at[pl.ds(sid * NPC, NPC)],
                    acc_hbm.at[cid, pl.ds(sid * NPC, NPC)])


def _final_body(acc_ref, o_ref):
    p = acc_ref[0, :N] + acc_ref[1, :N]
    z = p[:, MUL:MUL + 1]
    o_ref[...] = p[:, :MUL] * lax.rsqrt(jnp.maximum(z, 1e-30))


def kernel(x, edge_index, edge_attr, edge_len_embbed, edge_len,
           Wq, Wk1, Wk2, Wv1, Wv2, Wdot):
    f32 = jnp.float32
    w2k = Wk2.reshape(HID, D_IN, MUL).transpose(1, 2, 0).reshape(D_IN, MUL * HID)
    w2v = Wv2.reshape(HID, D_IN, MUL).transpose(1, 2, 0).reshape(D_IN, MUL * HID)
    w2 = jnp.concatenate([w2k, w2v], axis=1) * (1.0 / (np.sqrt(HID) * np.sqrt(D_IN)))
    wk1 = Wk1 * (1.0 / np.sqrt(BASIS))
    wv1 = Wv1 * (1.0 / np.sqrt(BASIS))
    wq = Wq * (1.0 / (np.sqrt(D_IN) * float(MUL)))
    sel = (jnp.arange(MUL * HID, dtype=jnp.int32)[:, None] // HID
           == jnp.arange(MUL, dtype=jnp.int32)[None, :]).astype(f32)
    sh0 = edge_attr[:, 0:1]
    el = edge_len[:, None]
    zeros = jnp.zeros((NPAD, OW), f32)
    src = edge_index[0]
    dst = edge_index[1]

    mesh = plsc.VectorSubcoreMesh(core_axis_name="c", subcore_axis_name="s")
    gather = pl.kernel(
        _gather_body,
        out_type=[jax.ShapeDtypeStruct((E, D_IN), f32),
                  jax.ShapeDtypeStruct((E, D_IN), f32)],
        mesh=mesh,
        scratch_types=[pltpu.VMEM((EPW,), jnp.int32),
                       pltpu.VMEM((EPW,), jnp.int32),
                       pltpu.VMEM((GC, D_IN), f32),
                       pltpu.VMEM((GC, D_IN), f32),
                       pltpu.VMEM((GC, D_IN), f32),
                       pltpu.VMEM((GC, D_IN), f32),
                       pltpu.SemaphoreType.DMA,
                       pltpu.SemaphoreType.DMA,
                       pltpu.SemaphoreType.DMA,
                       pltpu.SemaphoreType.DMA],
    )
    xg, xd = gather(src, dst, x)

    cur = lambda i: (jnp.minimum(i, GRID - 1), 0)
    prev = lambda i: (jnp.maximum(i, 1) - 1, 0)
    ewsv = pl.pallas_call(
        _edge_body,
        grid=(GRID + 1,),
        in_specs=[
            pl.BlockSpec((T, D_IN), cur),
            pl.BlockSpec((T, D_IN), cur),
            pl.BlockSpec((T, BASIS), cur),
            pl.BlockSpec((T, 1), cur),
            pl.BlockSpec((T, 1), prev),
            pl.BlockSpec((D_IN, 2 * MUL * HID), lambda i: (0, 0)),
            pl.BlockSpec((BASIS, 2 * HID), lambda i: (0, 0)),
            pl.BlockSpec((MUL * HID, MUL), lambda i: (0, 0)),
            pl.BlockSpec((D_IN, MUL), lambda i: (0, 0)),
        ],
        out_specs=pl.BlockSpec((T, OW), prev),
        out_shape=jax.ShapeDtypeStruct((E, OW), f32),
        scratch_shapes=[pltpu.VMEM((T, MUL), f32),
                        pltpu.VMEM((T, 1), f32)],
    )(xg, xd, edge_len_embbed, sh0, el, w2,
      jnp.concatenate([wk1, wv1], axis=1), sel, jnp.dot(wq, Wdot))

    scatter = pl.kernel(
        _scatter_body,
        out_type=jax.ShapeDtypeStruct((NC, NPAD, OW), f32),
        mesh=mesh,
        scratch_types=[pltpu.VMEM((EPW // SC_CHUNK, 1, SC_CHUNK), jnp.int32),
                       pltpu.VMEM((SC_CHUNK, OW), f32),
                       pltpu.VMEM((SC_CHUNK, OW), f32),
                       pltpu.VMEM_SHARED((NPAD, OW), f32),
                       pltpu.SemaphoreType.DMA,
                       pltpu.SemaphoreType.DMA],
    )
    acc = scatter(dst.reshape(E // SC_CHUNK, 1, SC_CHUNK), ewsv, zeros)

    out = pl.pallas_call(
        _final_body,
        out_shape=jax.ShapeDtypeStruct((N, MUL), f32),
    )(acc)
    return out
